# baseline (speedup 1.0000x reference)
import functools

import jax
import jax.numpy as jnp
from jax import lax
from jax.experimental import pallas as pl
from jax.experimental.pallas import tpu as pltpu
from jax.experimental.pallas import tpu_sc as plsc

N = 10000
IN_DIM = 128
NPAD = 10240
E = 320000
EPAD = 327680
NTILES = 32
NB = 80
BE = 128
RPT = NPAD // NTILES
HID = 16
DUMMY = 10000

_MESH = plsc.VectorSubcoreMesh(core_axis_name="c", subcore_axis_name="s")


DW = 8


def _deg_body(dst_hbm, zo_hbm, out_hbm, dst_v, zo_v, acc):
    cid = lax.axis_index("c")
    sid = lax.axis_index("s")
    wid = cid * 16 + sid

    pltpu.sync_copy(zo_hbm, zo_v)
    pltpu.sync_copy(zo_v.at[pl.ds(BE, RPT)], acc.at[pl.ds(sid * RPT, RPT)])
    plsc.subcore_barrier()

    pltpu.sync_copy(dst_hbm.at[wid], dst_v)

    def _scat(k, carry):
        pltpu.sync_copy(zo_v.at[pl.ds(0, BE)], acc.at[dst_v.at[k]], add=True)
        return carry

    lax.fori_loop(0, NB, _scat, 0)
    plsc.subcore_barrier()
    pltpu.sync_copy(acc.at[pl.ds(sid * RPT, RPT)],
                    out_hbm.at[pl.ds(cid * NPAD + sid * RPT, RPT)])


_deg_call = functools.partial(
    pl.kernel,
    out_type=jax.ShapeDtypeStruct((2 * NPAD, DW), jnp.float32),
    mesh=_MESH,
    compiler_params=pltpu.CompilerParams(use_tc_tiling_on_sc=False),
    scratch_types=[
        pltpu.VMEM((NB, BE), jnp.int32),
        pltpu.VMEM((BE + RPT, DW), jnp.float32),
        pltpu.VMEM_SHARED((NPAD, DW), jnp.float32),
    ],
)(_deg_body)


NBUF = 4


def _prop_body(yt_hbm, src_hbm, dst_hbm, out_hbm, src_v, dst_v, rows_v, zbuf,
               acc, s0, s1, s2, s3):
    cid = lax.axis_index("c")
    sid = lax.axis_index("s")
    wid = cid * 16 + sid
    sems = (s0, s1, s2, s3)

    def _fill_zero(i, carry):
        zbuf[i, :] = jnp.zeros((16,), jnp.float32)
        return carry

    lax.fori_loop(0, RPT, _fill_zero, 0)
    pltpu.sync_copy(zbuf, acc.at[pl.ds(sid * RPT, RPT)])
    plsc.subcore_barrier()

    pltpu.sync_copy(src_hbm.at[wid], src_v)
    pltpu.sync_copy(dst_hbm.at[wid], dst_v)

    for b in range(NBUF - 1):
        pltpu.async_copy(yt_hbm.at[src_v.at[b]], rows_v.at[b], sems[b])

    def _edge_group(g, carry):
        for b in range(NBUF):
            k = g * NBUF + b
            pltpu.make_async_copy(yt_hbm.at[src_v.at[0]], rows_v.at[b],
                                  sems[b]).wait()
            pltpu.sync_copy(rows_v.at[b], acc.at[dst_v.at[k]], add=True)
            nxt = k + NBUF - 1
            nb = (b + NBUF - 1) % NBUF

            @pl.when(nxt < NB)
            def _():
                pltpu.async_copy(yt_hbm.at[src_v.at[nxt]],
                                 rows_v.at[nb], sems[nb])

        return carry

    lax.fori_loop(0, NB // NBUF, _edge_group, 0)
    plsc.subcore_barrier()
    pltpu.sync_copy(acc.at[pl.ds(sid * RPT, RPT)],
                    out_hbm.at[pl.ds(cid * NPAD + sid * RPT, RPT)])


_prop_call = functools.partial(
    pl.kernel,
    out_type=jax.ShapeDtypeStruct((2 * NPAD, HID), jnp.float32),
    mesh=_MESH,
    compiler_params=pltpu.CompilerParams(use_tc_tiling_on_sc=False),
    scratch_types=[
        pltpu.VMEM((NB, BE), jnp.int32),
        pltpu.VMEM((NB, BE), jnp.int32),
        pltpu.VMEM((NBUF, BE, HID), jnp.float32),
        pltpu.VMEM((RPT, HID), jnp.float32),
        pltpu.VMEM_SHARED((NPAD, HID), jnp.float32),
        pltpu.SemaphoreType.DMA,
        pltpu.SemaphoreType.DMA,
        pltpu.SemaphoreType.DMA,
        pltpu.SemaphoreType.DMA,
    ],
)(_prop_body)


def _tc1_body(x_ref, w_ref, d0_ref, d1_ref, yt_ref, dinv_ref):
    deg = d0_ref[:, :1] + d1_ref[:, :1] + 1.0
    dinv = jnp.broadcast_to(lax.rsqrt(deg), (NPAD, HID))
    dinv_ref[...] = dinv
    xt = jnp.dot(x_ref[...], w_ref[...], preferred_element_type=jnp.float32)
    yt_ref[:N, :] = xt * dinv[:N, :]
    yt_ref[N:, :] = jnp.zeros((NPAD - N, HID), jnp.float32)


def _tc2_body(s0_ref, s1_ref, yt_ref, dinv_ref, w_ref, b_ref, out_ref):
    dinv = dinv_ref[...]
    h = jnp.maximum(dinv * (s0_ref[...] + s1_ref[...] + yt_ref[...])
                    + b_ref[...], 0.0)
    out_ref[...] = jnp.dot(h, w_ref[...],
                           preferred_element_type=jnp.float32) * dinv


def _tc3_body(s0_ref, s1_ref, yt_ref, dinv_ref, b_ref, out_ref):
    out_ref[...] = (dinv_ref[...] * (s0_ref[...] + s1_ref[...] + yt_ref[...])
                    + b_ref[...])


def _half_specs(minor):
    return [pl.BlockSpec((NPAD, minor), lambda i: (0, 0)),
            pl.BlockSpec((NPAD, minor), lambda i: (1, 0))]


def kernel(x, edge_index, W1, b1, W2, b2):
    src = edge_index[0]
    dst = edge_index[1]
    pad = jnp.full((EPAD - E,), DUMMY, jnp.int32)
    src3 = jnp.concatenate([src, pad]).reshape(NTILES, NB, BE)
    dst3 = jnp.concatenate([dst, pad]).reshape(NTILES, NB, BE)
    W2p = jnp.pad(W2, ((0, 0), (0, HID - W2.shape[1])))
    b1r = b1.reshape(1, HID)
    b2r = jnp.pad(b2, (0, HID - b2.shape[0])).reshape(1, HID)

    zo = jnp.concatenate([jnp.ones((BE, DW), jnp.float32),
                          jnp.zeros((RPT, DW), jnp.float32)])
    degp = _deg_call(dst3, zo)

    yt1, dinv = pl.pallas_call(
        _tc1_body,
        out_shape=(jax.ShapeDtypeStruct((NPAD, HID), jnp.float32),
                   jax.ShapeDtypeStruct((NPAD, HID), jnp.float32)),
    )(x, W1, degp[:NPAD], degp[NPAD:])

    s1 = _prop_call(yt1, src3, dst3)

    yt2 = pl.pallas_call(
        _tc2_body,
        out_shape=jax.ShapeDtypeStruct((NPAD, HID), jnp.float32),
    )(s1[:NPAD], s1[NPAD:], yt1, dinv, W2p, b1r)

    s2 = _prop_call(yt2, src3, dst3)

    out = pl.pallas_call(
        _tc3_body,
        out_shape=jax.ShapeDtypeStruct((NPAD, HID), jnp.float32),
    )(s2[:NPAD], s2[NPAD:], yt2, dinv, b2r)

    return out[:N, :W2.shape[1]]


# speedup vs baseline: 1.4718x; 1.4718x over previous
import functools

import jax
import jax.numpy as jnp
from jax import lax
from jax.experimental import pallas as pl
from jax.experimental.pallas import tpu as pltpu
from jax.experimental.pallas import tpu_sc as plsc

N = 10000
IN_DIM = 128
NPAD = 10240
E = 320000
EPAD = 327680
NTILES = 32
NB = 80
BE = 128
RPT = NPAD // NTILES
HID = 16
DUMMY = 10000

_MESH = plsc.VectorSubcoreMesh(core_axis_name="c", subcore_axis_name="s")


DW = 8


def _deg_body(dst_hbm, zo_hbm, out0_hbm, out1_hbm, dst_v, zo_v, acc):
    cid = lax.axis_index("c")
    sid = lax.axis_index("s")
    wid = cid * 16 + sid

    pltpu.sync_copy(zo_hbm, zo_v)
    pltpu.sync_copy(zo_v.at[pl.ds(BE, RPT)], acc.at[pl.ds(sid * RPT, RPT)])
    plsc.subcore_barrier()

    pltpu.sync_copy(dst_hbm.at[wid], dst_v)

    def _scat(k, carry):
        pltpu.sync_copy(zo_v.at[pl.ds(0, BE)], acc.at[dst_v.at[k]], add=True)
        return carry

    lax.fori_loop(0, NB, _scat, 0)
    plsc.subcore_barrier()

    @pl.when(cid == 0)
    def _():
        pltpu.sync_copy(acc.at[pl.ds(sid * RPT, RPT)],
                        out0_hbm.at[pl.ds(sid * RPT, RPT)])

    @pl.when(cid == 1)
    def _():
        pltpu.sync_copy(acc.at[pl.ds(sid * RPT, RPT)],
                        out1_hbm.at[pl.ds(sid * RPT, RPT)])


_deg_call = functools.partial(
    pl.kernel,
    out_type=(jax.ShapeDtypeStruct((NPAD, DW), jnp.float32),
              jax.ShapeDtypeStruct((NPAD, DW), jnp.float32)),
    mesh=_MESH,
    compiler_params=pltpu.CompilerParams(use_tc_tiling_on_sc=False),
    scratch_types=[
        pltpu.VMEM((NB, BE), jnp.int32),
        pltpu.VMEM((BE + RPT, DW), jnp.float32),
        pltpu.VMEM_SHARED((NPAD, DW), jnp.float32),
    ],
)(_deg_body)


NBUF = 4


def _prop_body(yt_hbm, src_hbm, dst_hbm, out0_hbm, out1_hbm, src_v, dst_v,
               rows_v, zbuf, acc, s0, s1, s2, s3):
    cid = lax.axis_index("c")
    sid = lax.axis_index("s")
    wid = cid * 16 + sid
    sems = (s0, s1, s2, s3)

    def _fill_zero(i, carry):
        zbuf[i, :] = jnp.zeros((16,), jnp.float32)
        return carry

    lax.fori_loop(0, RPT, _fill_zero, 0)
    pltpu.sync_copy(zbuf, acc.at[pl.ds(sid * RPT, RPT)])
    plsc.subcore_barrier()

    pltpu.sync_copy(src_hbm.at[wid], src_v)
    pltpu.sync_copy(dst_hbm.at[wid], dst_v)

    for b in range(NBUF - 1):
        pltpu.async_copy(yt_hbm.at[src_v.at[b]], rows_v.at[b], sems[b])

    def _edge_group(g, carry):
        for b in range(NBUF):
            k = g * NBUF + b
            pltpu.make_async_copy(yt_hbm.at[src_v.at[0]], rows_v.at[b],
                                  sems[b]).wait()
            pltpu.sync_copy(rows_v.at[b], acc.at[dst_v.at[k]], add=True)
            nxt = k + NBUF - 1
            nb = (b + NBUF - 1) % NBUF

            @pl.when(nxt < NB)
            def _():
                pltpu.async_copy(yt_hbm.at[src_v.at[nxt]],
                                 rows_v.at[nb], sems[nb])

        return carry

    lax.fori_loop(0, NB // NBUF, _edge_group, 0)
    plsc.subcore_barrier()

    @pl.when(cid == 0)
    def _():
        pltpu.sync_copy(acc.at[pl.ds(sid * RPT, RPT)],
                        out0_hbm.at[pl.ds(sid * RPT, RPT)])

    @pl.when(cid == 1)
    def _():
        pltpu.sync_copy(acc.at[pl.ds(sid * RPT, RPT)],
                        out1_hbm.at[pl.ds(sid * RPT, RPT)])


_prop_call = functools.partial(
    pl.kernel,
    out_type=(jax.ShapeDtypeStruct((NPAD, HID), jnp.float32),
              jax.ShapeDtypeStruct((NPAD, HID), jnp.float32)),
    mesh=_MESH,
    compiler_params=pltpu.CompilerParams(use_tc_tiling_on_sc=False),
    scratch_types=[
        pltpu.VMEM((NB, BE), jnp.int32),
        pltpu.VMEM((NB, BE), jnp.int32),
        pltpu.VMEM((NBUF, BE, HID), jnp.float32),
        pltpu.VMEM((RPT, HID), jnp.float32),
        pltpu.VMEM_SHARED((NPAD, HID), jnp.float32),
        pltpu.SemaphoreType.DMA,
        pltpu.SemaphoreType.DMA,
        pltpu.SemaphoreType.DMA,
        pltpu.SemaphoreType.DMA,
    ],
)(_prop_body)


def _tc1_body(x_ref, w_ref, d0_ref, d1_ref, yt_ref, dinv_ref):
    deg = d0_ref[:, :1] + d1_ref[:, :1] + 1.0
    dinv = jnp.broadcast_to(lax.rsqrt(deg), (NPAD, HID))
    dinv_ref[...] = dinv
    xt = jnp.dot(x_ref[...], w_ref[...], preferred_element_type=jnp.float32)
    yt_ref[:N, :] = xt * dinv[:N, :]
    yt_ref[N:, :] = jnp.zeros((NPAD - N, HID), jnp.float32)


def _tc2_body(s0_ref, s1_ref, yt_ref, dinv_ref, w_ref, b_ref, out_ref):
    dinv = dinv_ref[...]
    h = jnp.maximum(dinv * (s0_ref[...] + s1_ref[...] + yt_ref[...])
                    + b_ref[...], 0.0)
    out_ref[...] = jnp.dot(h, w_ref[...],
                           preferred_element_type=jnp.float32) * dinv


def _tc3_body(s0_ref, s1_ref, yt_ref, dinv_ref, b_ref, out_ref):
    out_ref[...] = (dinv_ref[...] * (s0_ref[...] + s1_ref[...] + yt_ref[...])
                    + b_ref[...])


def _half_specs(minor):
    return [pl.BlockSpec((NPAD, minor), lambda i: (0, 0)),
            pl.BlockSpec((NPAD, minor), lambda i: (1, 0))]


def kernel(x, edge_index, W1, b1, W2, b2):
    src = edge_index[0]
    dst = edge_index[1]
    pad = N + jnp.arange(EPAD - E, dtype=jnp.int32) % (NPAD - N)
    src3 = jnp.concatenate([src, pad]).reshape(NTILES, NB, BE)
    dst3 = jnp.concatenate([dst, pad]).reshape(NTILES, NB, BE)
    W2p = jnp.pad(W2, ((0, 0), (0, HID - W2.shape[1])))
    b1r = b1.reshape(1, HID)
    b2r = jnp.pad(b2, (0, HID - b2.shape[0])).reshape(1, HID)

    zo = jnp.concatenate([jnp.ones((BE, DW), jnp.float32),
                          jnp.zeros((RPT, DW), jnp.float32)])
    deg0, deg1 = _deg_call(dst3, zo)

    yt1, dinv = pl.pallas_call(
        _tc1_body,
        out_shape=(jax.ShapeDtypeStruct((NPAD, HID), jnp.float32),
                   jax.ShapeDtypeStruct((NPAD, HID), jnp.float32)),
    )(x, W1, deg0, deg1)

    s1a, s1b = _prop_call(yt1, src3, dst3)

    yt2 = pl.pallas_call(
        _tc2_body,
        out_shape=jax.ShapeDtypeStruct((NPAD, HID), jnp.float32),
    )(s1a, s1b, yt1, dinv, W2p, b1r)

    s2a, s2b = _prop_call(yt2, src3, dst3)

    out = pl.pallas_call(
        _tc3_body,
        out_shape=jax.ShapeDtypeStruct((NPAD, HID), jnp.float32),
    )(s2a, s2b, yt2, dinv, b2r)

    return out[:N, :W2.shape[1]]


# speedup vs baseline: 1.5619x; 1.0612x over previous
import functools

import jax
import jax.numpy as jnp
from jax import lax
from jax.experimental import pallas as pl
from jax.experimental.pallas import tpu as pltpu
from jax.experimental.pallas import tpu_sc as plsc

N = 10000
IN_DIM = 128
NPAD = 10240
E = 320000
EPAD = 327680
NTILES = 32
NB = 80
BE = 128
RPT = NPAD // NTILES
HID = 16
DUMMY = 10000

_MESH = plsc.VectorSubcoreMesh(core_axis_name="c", subcore_axis_name="s")


DW = 8


def _deg_body(dst_hbm, zo_hbm, out0_hbm, out1_hbm, dst_v, zo_v, acc):
    cid = lax.axis_index("c")
    sid = lax.axis_index("s")
    wid = cid * 16 + sid

    pltpu.sync_copy(zo_hbm, zo_v)
    pltpu.sync_copy(zo_v.at[pl.ds(BE, RPT)], acc.at[pl.ds(sid * RPT, RPT)])
    plsc.subcore_barrier()

    pltpu.sync_copy(dst_hbm.at[wid], dst_v)

    def _scat(k, carry):
        pltpu.sync_copy(zo_v.at[pl.ds(0, BE)], acc.at[dst_v.at[k]], add=True)
        return carry

    lax.fori_loop(0, NB, _scat, 0)
    plsc.subcore_barrier()

    @pl.when(cid == 0)
    def _():
        pltpu.sync_copy(acc.at[pl.ds(sid * RPT, RPT)],
                        out0_hbm.at[pl.ds(sid * RPT, RPT)])

    @pl.when(cid == 1)
    def _():
        pltpu.sync_copy(acc.at[pl.ds(sid * RPT, RPT)],
                        out1_hbm.at[pl.ds(sid * RPT, RPT)])


_deg_call = functools.partial(
    pl.kernel,
    out_type=(jax.ShapeDtypeStruct((NPAD, DW), jnp.float32),
              jax.ShapeDtypeStruct((NPAD, DW), jnp.float32)),
    mesh=_MESH,
    compiler_params=pltpu.CompilerParams(use_tc_tiling_on_sc=False),
    scratch_types=[
        pltpu.VMEM((NB, BE), jnp.int32),
        pltpu.VMEM((BE + RPT, DW), jnp.float32),
        pltpu.VMEM_SHARED((NPAD, DW), jnp.float32),
    ],
)(_deg_body)


NBUF = 5


def _prop_body(yt_hbm, src_hbm, dst_hbm, out0_hbm, out1_hbm, src_v, dst_v,
               rows_v, zbuf, acc, s0, s1, s2, s3, s4):
    cid = lax.axis_index("c")
    sid = lax.axis_index("s")
    wid = cid * 16 + sid
    sems = (s0, s1, s2, s3, s4)

    def _fill_zero(i, carry):
        zbuf[i, :] = jnp.zeros((16,), jnp.float32)
        return carry

    lax.fori_loop(0, RPT, _fill_zero, 0)
    pltpu.sync_copy(zbuf, acc.at[pl.ds(sid * RPT, RPT)])
    plsc.subcore_barrier()

    pltpu.sync_copy(src_hbm.at[wid], src_v)
    pltpu.sync_copy(dst_hbm.at[wid], dst_v)

    for b in range(NBUF - 1):
        pltpu.async_copy(yt_hbm.at[src_v.at[b]], rows_v.at[b], sems[b])

    def _edge_group(g, carry):
        for b in range(NBUF):
            k = g * NBUF + b
            pltpu.make_async_copy(yt_hbm.at[src_v.at[0]], rows_v.at[b],
                                  sems[b]).wait()
            pltpu.sync_copy(rows_v.at[b], acc.at[dst_v.at[k]], add=True)
            nxt = k + NBUF - 1
            nb = (b + NBUF - 1) % NBUF

            @pl.when(nxt < NB)
            def _():
                pltpu.async_copy(yt_hbm.at[src_v.at[nxt]],
                                 rows_v.at[nb], sems[nb])

        return carry

    lax.fori_loop(0, NB // NBUF, _edge_group, 0)
    plsc.subcore_barrier()

    @pl.when(cid == 0)
    def _():
        pltpu.sync_copy(acc.at[pl.ds(sid * RPT, RPT)],
                        out0_hbm.at[pl.ds(sid * RPT, RPT)])

    @pl.when(cid == 1)
    def _():
        pltpu.sync_copy(acc.at[pl.ds(sid * RPT, RPT)],
                        out1_hbm.at[pl.ds(sid * RPT, RPT)])


_prop_call = functools.partial(
    pl.kernel,
    out_type=(jax.ShapeDtypeStruct((NPAD, HID), jnp.float32),
              jax.ShapeDtypeStruct((NPAD, HID), jnp.float32)),
    mesh=_MESH,
    compiler_params=pltpu.CompilerParams(use_tc_tiling_on_sc=False),
    scratch_types=[
        pltpu.VMEM((NB, BE), jnp.int32),
        pltpu.VMEM((NB, BE), jnp.int32),
        pltpu.VMEM((NBUF, BE, HID), jnp.float32),
        pltpu.VMEM((RPT, HID), jnp.float32),
        pltpu.VMEM_SHARED((NPAD, HID), jnp.float32),
        pltpu.SemaphoreType.DMA,
        pltpu.SemaphoreType.DMA,
        pltpu.SemaphoreType.DMA,
        pltpu.SemaphoreType.DMA,
        pltpu.SemaphoreType.DMA,
    ],
)(_prop_body)


def _tc1_body(x_ref, w_ref, d0_ref, d1_ref, yt_ref, dinv_ref):
    deg = d0_ref[:, :1] + d1_ref[:, :1] + 1.0
    dinv = jnp.broadcast_to(lax.rsqrt(deg), (NPAD, HID))
    dinv_ref[...] = dinv
    xt = jnp.dot(x_ref[...], w_ref[...], preferred_element_type=jnp.float32)
    yt_ref[:N, :] = xt * dinv[:N, :]
    yt_ref[N:, :] = jnp.zeros((NPAD - N, HID), jnp.float32)


def _tc2_body(s0_ref, s1_ref, yt_ref, dinv_ref, w_ref, b_ref, out_ref):
    dinv = dinv_ref[...]
    h = jnp.maximum(dinv * (s0_ref[...] + s1_ref[...] + yt_ref[...])
                    + b_ref[...], 0.0)
    out_ref[...] = jnp.dot(h, w_ref[...],
                           preferred_element_type=jnp.float32) * dinv


def _tc3_body(s0_ref, s1_ref, yt_ref, dinv_ref, b_ref, out_ref):
    out_ref[...] = (dinv_ref[...] * (s0_ref[...] + s1_ref[...] + yt_ref[...])
                    + b_ref[...])


def _half_specs(minor):
    return [pl.BlockSpec((NPAD, minor), lambda i: (0, 0)),
            pl.BlockSpec((NPAD, minor), lambda i: (1, 0))]


def kernel(x, edge_index, W1, b1, W2, b2):
    src = edge_index[0]
    dst = edge_index[1]
    pad = N + jnp.arange(EPAD - E, dtype=jnp.int32) % (NPAD - N)
    src3 = jnp.concatenate([src, pad]).reshape(NTILES, NB, BE)
    dst3 = jnp.concatenate([dst, pad]).reshape(NTILES, NB, BE)
    W2p = jnp.pad(W2, ((0, 0), (0, HID - W2.shape[1])))
    b1r = b1.reshape(1, HID)
    b2r = jnp.pad(b2, (0, HID - b2.shape[0])).reshape(1, HID)

    zo = jnp.concatenate([jnp.ones((BE, DW), jnp.float32),
                          jnp.zeros((RPT, DW), jnp.float32)])
    deg0, deg1 = _deg_call(dst3, zo)

    yt1, dinv = pl.pallas_call(
        _tc1_body,
        out_shape=(jax.ShapeDtypeStruct((NPAD, HID), jnp.float32),
                   jax.ShapeDtypeStruct((NPAD, HID), jnp.float32)),
    )(x, W1, deg0, deg1)

    s1a, s1b = _prop_call(yt1, src3, dst3)

    rb = pl.BlockSpec((NPAD // 8, HID), lambda i: (i, 0))
    wb16 = pl.BlockSpec((HID, HID), lambda i: (0, 0))
    wb1 = pl.BlockSpec((1, HID), lambda i: (0, 0))
    yt2 = pl.pallas_call(
        _tc2_body,
        grid=(8,),
        in_specs=[rb, rb, rb, rb, wb16, wb1],
        out_specs=rb,
        out_shape=jax.ShapeDtypeStruct((NPAD, HID), jnp.float32),
    )(s1a, s1b, yt1, dinv, W2p, b1r)

    s2a, s2b = _prop_call(yt2, src3, dst3)

    out = pl.pallas_call(
        _tc3_body,
        grid=(8,),
        in_specs=[rb, rb, rb, rb, wb1],
        out_specs=rb,
        out_shape=jax.ShapeDtypeStruct((NPAD, HID), jnp.float32),
    )(s2a, s2b, yt2, dinv, b2r)

    return out[:N, :W2.shape[1]]
